# baseline (device time: 15634 ns/iter reference)
import functools

import jax
import jax.numpy as jnp
from jax import lax
from jax.experimental import pallas as pl
from jax.experimental.pallas import tpu as pltpu

N_DEV = 16


def kernel(x):
    m, n = x.shape

    def body(x_ref, out_ref, send_ref, totals_ref, send_sems, recv_sems):
        my = lax.axis_index("i")

        totals_ref[...] = jnp.zeros_like(totals_ref)

        barrier_sem = pltpu.get_barrier_semaphore()
        for d in range(N_DEV):
            @pl.when(my != d)
            def _(d=d):
                pl.semaphore_signal(
                    barrier_sem, inc=1,
                    device_id=(d,), device_id_type=pl.DeviceIdType.MESH,
                )

        xv = x_ref[...]
        send_ref[...] = jnp.sum(xv, axis=0, keepdims=True)

        pl.semaphore_wait(barrier_sem, N_DEV - 1)

        for t in range(1, N_DEV):
            @pl.when(my < t)
            def _(t=t):
                rdma = pltpu.make_async_remote_copy(
                    src_ref=send_ref,
                    dst_ref=totals_ref.at[my],
                    send_sem=send_sems.at[t],
                    recv_sem=recv_sems.at[my],
                    device_id=t,
                    device_id_type=pl.DeviceIdType.LOGICAL,
                )
                rdma.start()

        row = lax.broadcasted_iota(jnp.int32, (m, m), 0)
        col = lax.broadcasted_iota(jnp.int32, (m, m), 1)
        tril = (row >= col).astype(jnp.bfloat16)
        loc = lax.dot_general(
            tril, xv.astype(jnp.bfloat16),
            dimension_numbers=(((1,), (0,)), ((), ())),
            preferred_element_type=jnp.float32,
        )

        for j in range(N_DEV - 1):
            @pl.when(j < my)
            def _(j=j):
                recv = pltpu.make_async_remote_copy(
                    src_ref=send_ref,
                    dst_ref=totals_ref.at[j],
                    send_sem=send_sems.at[j],
                    recv_sem=recv_sems.at[j],
                    device_id=0,
                    device_id_type=pl.DeviceIdType.LOGICAL,
                )
                recv.wait_recv()

        offset = jnp.sum(totals_ref[...], axis=0)
        out_ref[...] = loc + offset

        for t in range(1, N_DEV):
            @pl.when(my < t)
            def _(t=t):
                send = pltpu.make_async_remote_copy(
                    src_ref=send_ref,
                    dst_ref=totals_ref.at[0],
                    send_sem=send_sems.at[t],
                    recv_sem=recv_sems.at[0],
                    device_id=0,
                    device_id_type=pl.DeviceIdType.LOGICAL,
                )
                send.wait_send()

        @functools.partial(pl.run_scoped, sem2=pltpu.SemaphoreType.REGULAR)
        def _(sem2):
            for d in range(N_DEV):
                @pl.when(my != d)
                def _(d=d):
                    pl.semaphore_signal(
                        sem2, inc=1,
                        device_id=(d,), device_id_type=pl.DeviceIdType.MESH,
                    )
            pl.semaphore_wait(sem2, N_DEV - 1)

    return pl.pallas_call(
        body,
        out_shape=jax.ShapeDtypeStruct((m, n), x.dtype),
        in_specs=[pl.BlockSpec(memory_space=pltpu.VMEM)],
        out_specs=pl.BlockSpec(memory_space=pltpu.VMEM),
        scratch_shapes=[
            pltpu.VMEM((1, n), jnp.float32),
            pltpu.VMEM((N_DEV, 1, n), jnp.float32),
            pltpu.SemaphoreType.DMA((N_DEV,)),
            pltpu.SemaphoreType.DMA((N_DEV,)),
        ],
        compiler_params=pltpu.CompilerParams(collective_id=0),
    )(x)


# device time: 9837 ns/iter; 1.5893x vs baseline; 1.5893x over previous
import jax
import jax.numpy as jnp
from jax import lax
from jax.experimental import pallas as pl
from jax.experimental.pallas import tpu as pltpu

N_DEV = 16


def kernel(x):
    m, n = x.shape

    def body(x_ref, out_ref, send_ref, totals_ref, send_sems, recv_sems):
        my = lax.axis_index("i")

        totals_ref[...] = jnp.zeros_like(totals_ref)

        barrier_sem = pltpu.get_barrier_semaphore()
        for d in range(N_DEV - 1):
            @pl.when(d < my)
            def _(d=d):
                pl.semaphore_signal(
                    barrier_sem, inc=1,
                    device_id=(d,), device_id_type=pl.DeviceIdType.MESH,
                )

        xv = x_ref[...]
        send_ref[...] = jnp.sum(xv, axis=0, keepdims=True)

        for t in range(1, N_DEV):
            @pl.when(my < t)
            def _():
                pl.semaphore_wait(barrier_sem, 1)

        for t in range(1, N_DEV):
            @pl.when(my < t)
            def _(t=t):
                rdma = pltpu.make_async_remote_copy(
                    src_ref=send_ref,
                    dst_ref=totals_ref.at[my],
                    send_sem=send_sems.at[t],
                    recv_sem=recv_sems.at[my],
                    device_id=t,
                    device_id_type=pl.DeviceIdType.LOGICAL,
                )
                rdma.start()

        row = lax.broadcasted_iota(jnp.int32, (m, m), 0)
        col = lax.broadcasted_iota(jnp.int32, (m, m), 1)
        tril = (row >= col).astype(jnp.bfloat16)
        loc = lax.dot_general(
            tril, xv.astype(jnp.bfloat16),
            dimension_numbers=(((1,), (0,)), ((), ())),
            preferred_element_type=jnp.float32,
        )

        for j in range(N_DEV - 1):
            @pl.when(j < my)
            def _(j=j):
                recv = pltpu.make_async_remote_copy(
                    src_ref=send_ref,
                    dst_ref=totals_ref.at[j],
                    send_sem=send_sems.at[j],
                    recv_sem=recv_sems.at[j],
                    device_id=0,
                    device_id_type=pl.DeviceIdType.LOGICAL,
                )
                recv.wait_recv()

        offset = jnp.sum(totals_ref[...], axis=0)
        out_ref[...] = loc + offset

        for t in range(1, N_DEV):
            @pl.when(my < t)
            def _(t=t):
                send = pltpu.make_async_remote_copy(
                    src_ref=send_ref,
                    dst_ref=totals_ref.at[0],
                    send_sem=send_sems.at[t],
                    recv_sem=recv_sems.at[0],
                    device_id=0,
                    device_id_type=pl.DeviceIdType.LOGICAL,
                )
                send.wait_send()


    return pl.pallas_call(
        body,
        out_shape=jax.ShapeDtypeStruct((m, n), x.dtype),
        in_specs=[pl.BlockSpec(memory_space=pltpu.VMEM)],
        out_specs=pl.BlockSpec(memory_space=pltpu.VMEM),
        scratch_shapes=[
            pltpu.VMEM((1, n), jnp.float32),
            pltpu.VMEM((N_DEV, 1, n), jnp.float32),
            pltpu.SemaphoreType.DMA((N_DEV,)),
            pltpu.SemaphoreType.DMA((N_DEV,)),
        ],
        compiler_params=pltpu.CompilerParams(collective_id=0),
    )(x)
